# baseline (speedup 1.0000x reference)
import functools

import jax
import jax.numpy as jnp
from jax import lax
from jax.experimental import pallas as pl
from jax.experimental.pallas import tpu as pltpu
from jax.experimental.pallas import tpu_sc as plsc

_N = 10000
_M = 32
_AF = 128
_NF = 16
_EDGES = _N * _M
_OUT = 2 * _AF
_EPS = 1e-5

_NA = 4800
_NB = _N - _NA

_NC = 2
_NS = 16
_NW = _NC * _NS
_CH = 80
_NBUF = 5


@functools.cache
def _make_sc_gather(n_edges):
    per_w = n_edges // _NW
    ngrp = per_w // (_CH * _NBUF)
    assert per_w % (_CH * _NBUF) == 0
    mesh = plsc.VectorSubcoreMesh(core_axis_name="c", subcore_axis_name="s")

    @functools.partial(
        pl.kernel,
        out_type=jax.ShapeDtypeStruct((n_edges, _AF), jnp.float32),
        mesh=mesh,
        scratch_types=[
            pltpu.VMEM((per_w,), jnp.int32),
            pltpu.VMEM((2, _NBUF, _CH, _AF), jnp.float32),
            pltpu.SemaphoreType.DMA,
            pltpu.SemaphoreType.DMA,
        ],
    )
    def _sc_gather(table_hbm, idx_hbm, out_hbm, idx_v, rows_v, gsem, wsem):
        wid = lax.axis_index("s") * _NC + lax.axis_index("c")
        base = wid * per_w
        pltpu.sync_copy(idx_hbm.at[pl.ds(base, per_w)], idx_v)

        def fire_gathers(g, bank):
            for b in range(_NBUF):
                off = (g * _NBUF + b) * _CH
                pltpu.async_copy(
                    table_hbm.at[idx_v.at[pl.ds(off, _CH)]],
                    rows_v.at[bank, b], gsem)

        def drain_gathers(g, bank):
            for b in range(_NBUF):
                off = (g * _NBUF + b) * _CH
                pltpu.make_async_copy(
                    table_hbm.at[idx_v.at[pl.ds(off, _CH)]],
                    rows_v.at[bank, b], gsem).wait()

        def fire_writes(g, bank):
            for b in range(_NBUF):
                off = base + (g * _NBUF + b) * _CH
                pltpu.async_copy(
                    rows_v.at[bank, b], out_hbm.at[pl.ds(off, _CH)], wsem)

        def drain_writes(g, bank):
            for b in range(_NBUF):
                off = base + (g * _NBUF + b) * _CH
                pltpu.make_async_copy(
                    rows_v.at[bank, b], out_hbm.at[pl.ds(off, _CH)],
                    wsem).wait()

        fire_gathers(0, 0)

        def body(g, _):
            bank = lax.rem(g, 2)
            other = 1 - bank
            drain_gathers(g, bank)

            @pl.when(g > 0)
            def _():
                drain_writes(g - 1, other)

            @pl.when(g + 1 < ngrp)
            def _():
                fire_gathers(g + 1, other)

            fire_writes(g, bank)
            return 0

        lax.fori_loop(0, ngrp, body, 0)
        drain_writes(ngrp - 1, lax.rem(ngrp - 1, 2))

    return _sc_gather


_TILE = 200
_ET = _TILE * _M
_TILE_F = 1000


def _softplus(x):
    return jnp.log(1.0 + jnp.exp(x))


def _sigmoid(x):
    return 0.5 * jnp.tanh(0.5 * x) + 0.5


def _gated_tile(a, g, e, wst, wnt, wet, b_row):
    s_tile = jnp.dot(a, wst, preferred_element_type=jnp.float32) + b_row
    q = jnp.dot(g, wnt, preferred_element_type=jnp.float32)
    r = jnp.dot(e, wet, preferred_element_type=jnp.float32)
    gated = (q + r).reshape(_TILE, _M, _OUT) + s_tile[:, None, :]
    return gated.reshape(_ET, _OUT)


def _tc_stats(atom, gathered, nbr2, wst, wnt, wet, prm):
    n_atoms = atom.shape[0]
    nt = n_atoms // _TILE

    def body(atom_ref, g_ref, e_ref, wst_ref, wnt_ref, wet_ref,
             prm_ref, st1_ref, acc_ref):
        j = pl.program_id(0)
        gated = _gated_tile(atom_ref[...], g_ref[...], e_ref[...],
                            wst_ref[...], wnt_ref[...], wet_ref[...],
                            prm_ref[0:1, :])

        @pl.when(j == 0)
        def _():
            acc_ref[...] = jnp.zeros_like(acc_ref)

        acc_ref[0:1, :] += jnp.sum(gated, axis=0, keepdims=True)
        acc_ref[1:2, :] += jnp.sum(gated * gated, axis=0, keepdims=True)

        @pl.when(j == nt - 1)
        def _():
            st1_ref[...] = acc_ref[...]

    return pl.pallas_call(
        body,
        grid=(nt,),
        in_specs=[
            pl.BlockSpec((_TILE, _AF), lambda j: (j, 0)),
            pl.BlockSpec((_ET, _AF), lambda j: (j, 0)),
            pl.BlockSpec((_ET, _NF), lambda j: (j, 0)),
            pl.BlockSpec((_AF, _OUT), lambda j: (0, 0)),
            pl.BlockSpec((_AF, _OUT), lambda j: (0, 0)),
            pl.BlockSpec((_NF, _OUT), lambda j: (0, 0)),
            pl.BlockSpec((8, _OUT), lambda j: (0, 0)),
        ],
        out_specs=pl.BlockSpec((8, _OUT), lambda j: (0, 0)),
        out_shape=jax.ShapeDtypeStruct((8, _OUT), jnp.float32),
        scratch_shapes=[pltpu.VMEM((8, _OUT), jnp.float32)],
    )(atom, gathered, nbr2, wst, wnt, wet, prm)


def _tc_main(atom, gathered, nbr2, wst, wnt, wet, prm, st1a, st1b):
    n_atoms = atom.shape[0]
    nt = n_atoms // _TILE

    def body(atom_ref, g_ref, e_ref, wst_ref, wnt_ref, wet_ref,
             prm_ref, st1a_ref, st1b_ref, ns_ref, st2_ref, acc2_ref):
        j = pl.program_id(0)
        inv_e = 1.0 / _EDGES
        st1_sum = st1a_ref[0:1, :] + st1b_ref[0:1, :]
        st1_sq = st1a_ref[1:2, :] + st1b_ref[1:2, :]
        mean1 = st1_sum * inv_e
        var1 = st1_sq * inv_e - mean1 * mean1
        scale = prm_ref[1:2, :] * lax.rsqrt(var1 + _EPS)
        shift = prm_ref[2:3, :] - mean1 * scale
        x = _gated_tile(atom_ref[...], g_ref[...], e_ref[...],
                        wst_ref[...] * scale, wnt_ref[...] * scale,
                        wet_ref[...] * scale,
                        prm_ref[0:1, :] * scale + shift)

        filt = _sigmoid(x[:, :_AF])
        core = _softplus(x[:, _AF:])
        ns = jnp.sum((filt * core).reshape(_TILE, _M, _AF), axis=1)
        ns_ref[...] = ns

        @pl.when(j == 0)
        def _():
            acc2_ref[...] = jnp.zeros_like(acc2_ref)

        acc2_ref[0:1, :] += jnp.sum(ns, axis=0, keepdims=True)
        acc2_ref[1:2, :] += jnp.sum(ns * ns, axis=0, keepdims=True)

        @pl.when(j == nt - 1)
        def _():
            st2_ref[...] = acc2_ref[...]

    return pl.pallas_call(
        body,
        grid=(nt,),
        in_specs=[
            pl.BlockSpec((_TILE, _AF), lambda j: (j, 0)),
            pl.BlockSpec((_ET, _AF), lambda j: (j, 0)),
            pl.BlockSpec((_ET, _NF), lambda j: (j, 0)),
            pl.BlockSpec((_AF, _OUT), lambda j: (0, 0)),
            pl.BlockSpec((_AF, _OUT), lambda j: (0, 0)),
            pl.BlockSpec((_NF, _OUT), lambda j: (0, 0)),
            pl.BlockSpec((8, _OUT), lambda j: (0, 0)),
            pl.BlockSpec((8, _OUT), lambda j: (0, 0)),
            pl.BlockSpec((8, _OUT), lambda j: (0, 0)),
        ],
        out_specs=[
            pl.BlockSpec((_TILE, _AF), lambda j: (j, 0)),
            pl.BlockSpec((8, _AF), lambda j: (0, 0)),
        ],
        out_shape=[
            jax.ShapeDtypeStruct((n_atoms, _AF), jnp.float32),
            jax.ShapeDtypeStruct((8, _AF), jnp.float32),
        ],
        scratch_shapes=[pltpu.VMEM((8, _AF), jnp.float32)],
    )(atom, gathered, nbr2, wst, wnt, wet, prm, st1a, st1b)


def _tc_final(atom, nsa, nsb, st2a, st2b, prm2):
    def body(atom_ref, ns_ref, st2a_ref, st2b_ref, prm2_ref, out_ref):
        inv_n = 1.0 / _N
        mean2 = (st2a_ref[0:1, :] + st2b_ref[0:1, :]) * inv_n
        var2 = (st2a_ref[1:2, :] + st2b_ref[1:2, :]) * inv_n - mean2 * mean2
        scale = prm2_ref[0:1, :] * lax.rsqrt(var2 + _EPS)
        shift = prm2_ref[1:2, :] - mean2 * scale
        out_ref[...] = _softplus(atom_ref[...] + ns_ref[...] * scale + shift)

    ns = jnp.concatenate([nsa, nsb], axis=0)
    return pl.pallas_call(
        body,
        grid=(_N // _TILE_F,),
        in_specs=[
            pl.BlockSpec((_TILE_F, _AF), lambda j: (j, 0)),
            pl.BlockSpec((_TILE_F, _AF), lambda j: (j, 0)),
            pl.BlockSpec((8, _AF), lambda j: (0, 0)),
            pl.BlockSpec((8, _AF), lambda j: (0, 0)),
            pl.BlockSpec((8, _AF), lambda j: (0, 0)),
        ],
        out_specs=pl.BlockSpec((_TILE_F, _AF), lambda j: (j, 0)),
        out_shape=jax.ShapeDtypeStruct((_N, _AF), jnp.float32),
    )(atom, ns, st2a, st2b, prm2)


def kernel(atom_in_fea, nbr_fea, nbr_fea_idx, W_fc, b_fc,
           bn1_gamma, bn1_beta, bn2_gamma, bn2_beta):
    idx_flat = nbr_fea_idx.reshape(-1).astype(jnp.int32)
    idx_a = idx_flat[:_NA * _M]
    idx_b = idx_flat[_NA * _M:]
    ga = _make_sc_gather(_NA * _M)(atom_in_fea, idx_a)
    gb = _make_sc_gather(_NB * _M)(atom_in_fea, idx_b)

    nbr2 = nbr_fea.reshape(_EDGES, _NF)
    ea = nbr2[:_NA * _M]
    eb = nbr2[_NA * _M:]
    atom_a = atom_in_fea[:_NA]
    atom_b = atom_in_fea[_NA:]

    wst = W_fc[:, :_AF].T
    wnt = W_fc[:, _AF:2 * _AF].T
    wet = W_fc[:, 2 * _AF:].T
    prm = jnp.zeros((8, _OUT), jnp.float32)
    prm = prm.at[0].set(b_fc).at[1].set(bn1_gamma).at[2].set(bn1_beta)
    prm2 = jnp.zeros((8, _AF), jnp.float32)
    prm2 = prm2.at[0].set(bn2_gamma).at[1].set(bn2_beta)

    st1a = _tc_stats(atom_a, ga, ea, wst, wnt, wet, prm)
    st1b = _tc_stats(atom_b, gb, eb, wst, wnt, wet, prm)
    nsa, st2a = _tc_main(atom_a, ga, ea, wst, wnt, wet, prm, st1a, st1b)
    nsb, st2b = _tc_main(atom_b, gb, eb, wst, wnt, wet, prm, st1a, st1b)
    return _tc_final(atom_in_fea, nsa, nsb, st2a, st2b, prm2)


# speedup vs baseline: 1.1132x; 1.1132x over previous
import functools

import jax
import jax.numpy as jnp
from jax import lax
from jax.experimental import pallas as pl
from jax.experimental.pallas import tpu as pltpu
from jax.experimental.pallas import tpu_sc as plsc

_N = 10000
_M = 32
_AF = 128
_NF = 16
_EDGES = _N * _M
_OUT = 2 * _AF
_EPS = 1e-5

_NC = 2
_NS = 16
_NW = _NC * _NS
_CH = 80
_NBUF = 5


@functools.cache
def _make_sc_gather(n_edges):
    per_w = n_edges // _NW
    ngrp = per_w // (_CH * _NBUF)
    assert per_w % (_CH * _NBUF) == 0
    mesh = plsc.VectorSubcoreMesh(core_axis_name="c", subcore_axis_name="s")

    @functools.partial(
        pl.kernel,
        out_type=jax.ShapeDtypeStruct((n_edges, _AF), jnp.float32),
        mesh=mesh,
        scratch_types=[
            pltpu.VMEM((per_w,), jnp.int32),
            pltpu.VMEM((2, _NBUF, _CH, _AF), jnp.float32),
            pltpu.SemaphoreType.DMA,
            pltpu.SemaphoreType.DMA,
        ],
    )
    def _sc_gather(table_hbm, idx_hbm, out_hbm, idx_v, rows_v, gsem, wsem):
        wid = lax.axis_index("s") * _NC + lax.axis_index("c")
        base = wid * per_w
        pltpu.sync_copy(idx_hbm.at[pl.ds(base, per_w)], idx_v)

        def fire_gathers(g, bank):
            for b in range(_NBUF):
                off = (g * _NBUF + b) * _CH
                pltpu.async_copy(
                    table_hbm.at[idx_v.at[pl.ds(off, _CH)]],
                    rows_v.at[bank, b], gsem)

        def drain_gathers(g, bank):
            for b in range(_NBUF):
                off = (g * _NBUF + b) * _CH
                pltpu.make_async_copy(
                    table_hbm.at[idx_v.at[pl.ds(off, _CH)]],
                    rows_v.at[bank, b], gsem).wait()

        def fire_writes(g, bank):
            for b in range(_NBUF):
                off = base + (g * _NBUF + b) * _CH
                pltpu.async_copy(
                    rows_v.at[bank, b], out_hbm.at[pl.ds(off, _CH)], wsem)

        def drain_writes(g, bank):
            for b in range(_NBUF):
                off = base + (g * _NBUF + b) * _CH
                pltpu.make_async_copy(
                    rows_v.at[bank, b], out_hbm.at[pl.ds(off, _CH)],
                    wsem).wait()

        fire_gathers(0, 0)

        def body(g, _):
            bank = lax.rem(g, 2)
            other = 1 - bank
            drain_gathers(g, bank)

            @pl.when(g > 0)
            def _():
                drain_writes(g - 1, other)

            @pl.when(g + 1 < ngrp)
            def _():
                fire_gathers(g + 1, other)

            fire_writes(g, bank)
            return 0

        lax.fori_loop(0, ngrp, body, 0)
        drain_writes(ngrp - 1, lax.rem(ngrp - 1, 2))

    return _sc_gather


_TILE = 200
_ET = _TILE * _M
_TILE_F = 1000


def _softplus(x):
    return jnp.log(1.0 + jnp.exp(x))


def _sigmoid(x):
    return 0.5 * jnp.tanh(0.5 * x) + 0.5


def _gated_tile(a, g, e, wst, wnt, wet, b_row):
    s_tile = jnp.dot(a, wst, preferred_element_type=jnp.float32) + b_row
    q = jnp.dot(g, wnt, preferred_element_type=jnp.float32)
    r = jnp.dot(e, wet, preferred_element_type=jnp.float32)
    gated = (q + r).reshape(_TILE, _M, _OUT) + s_tile[:, None, :]
    return gated.reshape(_ET, _OUT)


def _tc_stats(atom, gathered, nbr2, wst, wnt, wet, prm):
    n_atoms = atom.shape[0]
    nt = n_atoms // _TILE

    def body(atom_ref, g_ref, e_ref, wst_ref, wnt_ref, wet_ref,
             prm_ref, st1_ref, acc_ref):
        j = pl.program_id(0)
        gated = _gated_tile(atom_ref[...], g_ref[...], e_ref[...],
                            wst_ref[...], wnt_ref[...], wet_ref[...],
                            prm_ref[0:1, :])

        @pl.when(j == 0)
        def _():
            acc_ref[...] = jnp.zeros_like(acc_ref)

        acc_ref[0:1, :] += jnp.sum(gated, axis=0, keepdims=True)
        acc_ref[1:2, :] += jnp.sum(gated * gated, axis=0, keepdims=True)

        @pl.when(j == nt - 1)
        def _():
            st1_ref[...] = acc_ref[...]

    return pl.pallas_call(
        body,
        grid=(nt,),
        in_specs=[
            pl.BlockSpec((_TILE, _AF), lambda j: (j, 0)),
            pl.BlockSpec((_ET, _AF), lambda j: (j, 0)),
            pl.BlockSpec((_ET, _NF), lambda j: (j, 0)),
            pl.BlockSpec((_AF, _OUT), lambda j: (0, 0)),
            pl.BlockSpec((_AF, _OUT), lambda j: (0, 0)),
            pl.BlockSpec((_NF, _OUT), lambda j: (0, 0)),
            pl.BlockSpec((8, _OUT), lambda j: (0, 0)),
        ],
        out_specs=pl.BlockSpec((8, _OUT), lambda j: (0, 0)),
        out_shape=jax.ShapeDtypeStruct((8, _OUT), jnp.float32),
        scratch_shapes=[pltpu.VMEM((8, _OUT), jnp.float32)],
    )(atom, gathered, nbr2, wst, wnt, wet, prm)


def _tc_main(atom, gathered, nbr2, wst, wnt, wet, prm, st1):
    n_atoms = atom.shape[0]
    nt = n_atoms // _TILE

    def body(atom_ref, g_ref, e_ref, wst_ref, wnt_ref, wet_ref,
             prm_ref, st1_ref, ns_ref, st2_ref, acc2_ref):
        j = pl.program_id(0)
        inv_e = 1.0 / _EDGES
        mean1 = st1_ref[0:1, :] * inv_e
        var1 = st1_ref[1:2, :] * inv_e - mean1 * mean1
        scale = prm_ref[1:2, :] * lax.rsqrt(var1 + _EPS)
        shift = prm_ref[2:3, :] - mean1 * scale
        x = _gated_tile(atom_ref[...], g_ref[...], e_ref[...],
                        wst_ref[...] * scale, wnt_ref[...] * scale,
                        wet_ref[...] * scale,
                        prm_ref[0:1, :] * scale + shift)

        filt = _sigmoid(x[:, :_AF])
        core = _softplus(x[:, _AF:])
        ns = jnp.sum((filt * core).reshape(_TILE, _M, _AF), axis=1)
        ns_ref[...] = ns

        @pl.when(j == 0)
        def _():
            acc2_ref[...] = jnp.zeros_like(acc2_ref)

        acc2_ref[0:1, :] += jnp.sum(ns, axis=0, keepdims=True)
        acc2_ref[1:2, :] += jnp.sum(ns * ns, axis=0, keepdims=True)

        @pl.when(j == nt - 1)
        def _():
            st2_ref[...] = acc2_ref[...]

    return pl.pallas_call(
        body,
        grid=(nt,),
        in_specs=[
            pl.BlockSpec((_TILE, _AF), lambda j: (j, 0)),
            pl.BlockSpec((_ET, _AF), lambda j: (j, 0)),
            pl.BlockSpec((_ET, _NF), lambda j: (j, 0)),
            pl.BlockSpec((_AF, _OUT), lambda j: (0, 0)),
            pl.BlockSpec((_AF, _OUT), lambda j: (0, 0)),
            pl.BlockSpec((_NF, _OUT), lambda j: (0, 0)),
            pl.BlockSpec((8, _OUT), lambda j: (0, 0)),
            pl.BlockSpec((8, _OUT), lambda j: (0, 0)),
        ],
        out_specs=[
            pl.BlockSpec((_TILE, _AF), lambda j: (j, 0)),
            pl.BlockSpec((8, _AF), lambda j: (0, 0)),
        ],
        out_shape=[
            jax.ShapeDtypeStruct((n_atoms, _AF), jnp.float32),
            jax.ShapeDtypeStruct((8, _AF), jnp.float32),
        ],
        scratch_shapes=[pltpu.VMEM((8, _AF), jnp.float32)],
    )(atom, gathered, nbr2, wst, wnt, wet, prm, st1)


def _tc_final(atom, ns, st2, prm2):
    def body(atom_ref, ns_ref, st2_ref, prm2_ref, out_ref):
        inv_n = 1.0 / _N
        mean2 = st2_ref[0:1, :] * inv_n
        var2 = st2_ref[1:2, :] * inv_n - mean2 * mean2
        scale = prm2_ref[0:1, :] * lax.rsqrt(var2 + _EPS)
        shift = prm2_ref[1:2, :] - mean2 * scale
        out_ref[...] = _softplus(atom_ref[...] + ns_ref[...] * scale + shift)

    return pl.pallas_call(
        body,
        grid=(_N // _TILE_F,),
        in_specs=[
            pl.BlockSpec((_TILE_F, _AF), lambda j: (j, 0)),
            pl.BlockSpec((_TILE_F, _AF), lambda j: (j, 0)),
            pl.BlockSpec((8, _AF), lambda j: (0, 0)),
            pl.BlockSpec((8, _AF), lambda j: (0, 0)),
        ],
        out_specs=pl.BlockSpec((_TILE_F, _AF), lambda j: (j, 0)),
        out_shape=jax.ShapeDtypeStruct((_N, _AF), jnp.float32),
    )(atom, ns, st2, prm2)


def kernel(atom_in_fea, nbr_fea, nbr_fea_idx, W_fc, b_fc,
           bn1_gamma, bn1_beta, bn2_gamma, bn2_beta):
    idx_flat = nbr_fea_idx.reshape(-1).astype(jnp.int32)
    gathered = _make_sc_gather(_EDGES)(atom_in_fea, idx_flat)

    nbr2 = nbr_fea.reshape(_EDGES, _NF)
    wst = W_fc[:, :_AF].T
    wnt = W_fc[:, _AF:2 * _AF].T
    wet = W_fc[:, 2 * _AF:].T
    prm = jnp.zeros((8, _OUT), jnp.float32)
    prm = prm.at[0].set(b_fc).at[1].set(bn1_gamma).at[2].set(bn1_beta)
    prm2 = jnp.zeros((8, _AF), jnp.float32)
    prm2 = prm2.at[0].set(bn2_gamma).at[1].set(bn2_beta)

    st1 = _tc_stats(atom_in_fea, gathered, nbr2, wst, wnt, wet, prm)
    ns, st2 = _tc_main(atom_in_fea, gathered, nbr2, wst, wnt, wet, prm, st1)
    return _tc_final(atom_in_fea, ns, st2, prm2)


# speedup vs baseline: 1.1545x; 1.0371x over previous
import functools

import jax
import jax.numpy as jnp
from jax import lax
from jax.experimental import pallas as pl
from jax.experimental.pallas import tpu as pltpu
from jax.experimental.pallas import tpu_sc as plsc

_N = 10000
_M = 32
_AF = 128
_NF = 16
_EDGES = _N * _M
_OUT = 2 * _AF
_EPS = 1e-5

_NC = 2
_NS = 16
_NW = _NC * _NS
_CH = 80
_NBUF = 5


@functools.cache
def _make_sc_gather(n_edges):
    per_w = n_edges // _NW
    ngrp = per_w // (_CH * _NBUF)
    assert per_w % (_CH * _NBUF) == 0
    mesh = plsc.VectorSubcoreMesh(core_axis_name="c", subcore_axis_name="s")

    @functools.partial(
        pl.kernel,
        out_type=jax.ShapeDtypeStruct((n_edges, _AF), jnp.float32),
        mesh=mesh,
        scratch_types=[
            pltpu.VMEM((per_w,), jnp.int32),
            pltpu.VMEM((2, _NBUF, _CH, _AF), jnp.float32),
            pltpu.SemaphoreType.DMA,
            pltpu.SemaphoreType.DMA,
        ],
    )
    def _sc_gather(table_hbm, idx_hbm, out_hbm, idx_v, rows_v, gsem, wsem):
        wid = lax.axis_index("s") * _NC + lax.axis_index("c")
        base = wid * per_w
        pltpu.sync_copy(idx_hbm.at[pl.ds(base, per_w)], idx_v)

        def fire_gathers(g, bank):
            for b in range(_NBUF):
                off = (g * _NBUF + b) * _CH
                pltpu.async_copy(
                    table_hbm.at[idx_v.at[pl.ds(off, _CH)]],
                    rows_v.at[bank, b], gsem)

        def drain_gathers(g, bank):
            for b in range(_NBUF):
                off = (g * _NBUF + b) * _CH
                pltpu.make_async_copy(
                    table_hbm.at[idx_v.at[pl.ds(off, _CH)]],
                    rows_v.at[bank, b], gsem).wait()

        def fire_writes(g, bank):
            for b in range(_NBUF):
                off = base + (g * _NBUF + b) * _CH
                pltpu.async_copy(
                    rows_v.at[bank, b], out_hbm.at[pl.ds(off, _CH)], wsem)

        def drain_writes(g, bank):
            for b in range(_NBUF):
                off = base + (g * _NBUF + b) * _CH
                pltpu.make_async_copy(
                    rows_v.at[bank, b], out_hbm.at[pl.ds(off, _CH)],
                    wsem).wait()

        fire_gathers(0, 0)

        def body(g, _):
            bank = lax.rem(g, 2)
            other = 1 - bank
            drain_gathers(g, bank)

            @pl.when(g > 0)
            def _():
                drain_writes(g - 1, other)

            @pl.when(g + 1 < ngrp)
            def _():
                fire_gathers(g + 1, other)

            fire_writes(g, bank)
            return 0

        lax.fori_loop(0, ngrp, body, 0)
        drain_writes(ngrp - 1, lax.rem(ngrp - 1, 2))

    return _sc_gather


_TILE = 400
_ET = _TILE * _M
_TILE_F = 1000


def _softplus(x):
    return jnp.log(1.0 + jnp.exp(x))


def _sigmoid(x):
    return 0.5 * jnp.tanh(0.5 * x) + 0.5


def _gated_tile(a, g, e, wst, wnt, wet, b_row):
    s_tile = jnp.dot(a, wst, preferred_element_type=jnp.float32) + b_row
    q = jnp.dot(g, wnt, preferred_element_type=jnp.float32)
    r = jnp.dot(e, wet, preferred_element_type=jnp.float32)
    gated = (q + r).reshape(_TILE, _M, _OUT) + s_tile[:, None, :]
    return gated.reshape(_ET, _OUT)


def _tc_stats(atom, gathered, nbr2, wst, wnt, wet, prm):
    n_atoms = atom.shape[0]
    nt = n_atoms // _TILE

    def body(atom_ref, g_ref, e_ref, wst_ref, wnt_ref, wet_ref,
             prm_ref, st1_ref, acc_ref):
        j = pl.program_id(0)
        gated = _gated_tile(atom_ref[...], g_ref[...], e_ref[...],
                            wst_ref[...], wnt_ref[...], wet_ref[...],
                            prm_ref[0:1, :])

        @pl.when(j == 0)
        def _():
            acc_ref[...] = jnp.zeros_like(acc_ref)

        acc_ref[0:1, :] += jnp.sum(gated, axis=0, keepdims=True)
        acc_ref[1:2, :] += jnp.sum(gated * gated, axis=0, keepdims=True)

        @pl.when(j == nt - 1)
        def _():
            st1_ref[...] = acc_ref[...]

    return pl.pallas_call(
        body,
        grid=(nt,),
        in_specs=[
            pl.BlockSpec((_TILE, _AF), lambda j: (j, 0)),
            pl.BlockSpec((_ET, _AF), lambda j: (j, 0)),
            pl.BlockSpec((_ET, _NF), lambda j: (j, 0)),
            pl.BlockSpec((_AF, _OUT), lambda j: (0, 0)),
            pl.BlockSpec((_AF, _OUT), lambda j: (0, 0)),
            pl.BlockSpec((_NF, _OUT), lambda j: (0, 0)),
            pl.BlockSpec((8, _OUT), lambda j: (0, 0)),
        ],
        out_specs=pl.BlockSpec((8, _OUT), lambda j: (0, 0)),
        out_shape=jax.ShapeDtypeStruct((8, _OUT), jnp.float32),
        scratch_shapes=[pltpu.VMEM((8, _OUT), jnp.float32)],
    )(atom, gathered, nbr2, wst, wnt, wet, prm)


def _tc_main(atom, gathered, nbr2, wst, wnt, wet, prm, st1):
    n_atoms = atom.shape[0]
    nt = n_atoms // _TILE

    def body(atom_ref, g_ref, e_ref, wst_ref, wnt_ref, wet_ref,
             prm_ref, st1_ref, ns_ref, st2_ref, acc2_ref):
        j = pl.program_id(0)
        inv_e = 1.0 / _EDGES
        mean1 = st1_ref[0:1, :] * inv_e
        var1 = st1_ref[1:2, :] * inv_e - mean1 * mean1
        scale = prm_ref[1:2, :] * lax.rsqrt(var1 + _EPS)
        shift = prm_ref[2:3, :] - mean1 * scale
        x = _gated_tile(atom_ref[...], g_ref[...], e_ref[...],
                        wst_ref[...] * scale, wnt_ref[...] * scale,
                        wet_ref[...] * scale,
                        prm_ref[0:1, :] * scale + shift)

        filt = _sigmoid(x[:, :_AF])
        core = _softplus(x[:, _AF:])
        ns = jnp.sum((filt * core).reshape(_TILE, _M, _AF), axis=1)
        ns_ref[...] = ns

        @pl.when(j == 0)
        def _():
            acc2_ref[...] = jnp.zeros_like(acc2_ref)

        acc2_ref[0:1, :] += jnp.sum(ns, axis=0, keepdims=True)
        acc2_ref[1:2, :] += jnp.sum(ns * ns, axis=0, keepdims=True)

        @pl.when(j == nt - 1)
        def _():
            st2_ref[...] = acc2_ref[...]

    return pl.pallas_call(
        body,
        grid=(nt,),
        in_specs=[
            pl.BlockSpec((_TILE, _AF), lambda j: (j, 0)),
            pl.BlockSpec((_ET, _AF), lambda j: (j, 0)),
            pl.BlockSpec((_ET, _NF), lambda j: (j, 0)),
            pl.BlockSpec((_AF, _OUT), lambda j: (0, 0)),
            pl.BlockSpec((_AF, _OUT), lambda j: (0, 0)),
            pl.BlockSpec((_NF, _OUT), lambda j: (0, 0)),
            pl.BlockSpec((8, _OUT), lambda j: (0, 0)),
            pl.BlockSpec((8, _OUT), lambda j: (0, 0)),
        ],
        out_specs=[
            pl.BlockSpec((_TILE, _AF), lambda j: (j, 0)),
            pl.BlockSpec((8, _AF), lambda j: (0, 0)),
        ],
        out_shape=[
            jax.ShapeDtypeStruct((n_atoms, _AF), jnp.float32),
            jax.ShapeDtypeStruct((8, _AF), jnp.float32),
        ],
        scratch_shapes=[pltpu.VMEM((8, _AF), jnp.float32)],
    )(atom, gathered, nbr2, wst, wnt, wet, prm, st1)


def _tc_final(atom, ns, st2, prm2):
    def body(atom_ref, ns_ref, st2_ref, prm2_ref, out_ref):
        inv_n = 1.0 / _N
        mean2 = st2_ref[0:1, :] * inv_n
        var2 = st2_ref[1:2, :] * inv_n - mean2 * mean2
        scale = prm2_ref[0:1, :] * lax.rsqrt(var2 + _EPS)
        shift = prm2_ref[1:2, :] - mean2 * scale
        out_ref[...] = _softplus(atom_ref[...] + ns_ref[...] * scale + shift)

    return pl.pallas_call(
        body,
        grid=(_N // _TILE_F,),
        in_specs=[
            pl.BlockSpec((_TILE_F, _AF), lambda j: (j, 0)),
            pl.BlockSpec((_TILE_F, _AF), lambda j: (j, 0)),
            pl.BlockSpec((8, _AF), lambda j: (0, 0)),
            pl.BlockSpec((8, _AF), lambda j: (0, 0)),
        ],
        out_specs=pl.BlockSpec((_TILE_F, _AF), lambda j: (j, 0)),
        out_shape=jax.ShapeDtypeStruct((_N, _AF), jnp.float32),
    )(atom, ns, st2, prm2)


def kernel(atom_in_fea, nbr_fea, nbr_fea_idx, W_fc, b_fc,
           bn1_gamma, bn1_beta, bn2_gamma, bn2_beta):
    idx_flat = nbr_fea_idx.reshape(-1).astype(jnp.int32)
    gathered = _make_sc_gather(_EDGES)(atom_in_fea, idx_flat)

    nbr2 = nbr_fea.reshape(_EDGES, _NF)
    wst = W_fc[:, :_AF].T
    wnt = W_fc[:, _AF:2 * _AF].T
    wet = W_fc[:, 2 * _AF:].T
    prm = jnp.zeros((8, _OUT), jnp.float32)
    prm = prm.at[0].set(b_fc).at[1].set(bn1_gamma).at[2].set(bn1_beta)
    prm2 = jnp.zeros((8, _AF), jnp.float32)
    prm2 = prm2.at[0].set(bn2_gamma).at[1].set(bn2_beta)

    st1 = _tc_stats(atom_in_fea, gathered, nbr2, wst, wnt, wet, prm)
    ns, st2 = _tc_main(atom_in_fea, gathered, nbr2, wst, wnt, wet, prm, st1)
    return _tc_final(atom_in_fea, ns, st2, prm2)
